# baseline (speedup 1.0000x reference)
import jax
import jax.numpy as jnp
from jax.experimental import pallas as pl

B, C = 1024, 7
CC = C * C
Q = B * CC
BI = 32


def _body(xrow_ref, wrow_ref, xrep_ref, pat_ref, out_ref):
    xr = xrow_ref[...]
    w = wrow_ref[...]
    p0 = jnp.abs(xr) * w
    d = (jnp.abs(xr - 1.0) - jnp.abs(xr)) * w
    xi = xrep_ref[...]
    idx = (xi * (xi >= 0).astype(xi.dtype)).astype(jnp.int32)
    lio = jax.lax.broadcasted_iota(jnp.int32, (BI, CC), 1)
    oh = (idx == lio % C).astype(jnp.bfloat16)
    m = jax.lax.dot_general(
        oh, pat_ref[...],
        dimension_numbers=(((1,), (0,)), ((), ())),
        preferred_element_type=jnp.float32,
    )
    out_ref[...] = p0 + m * d


def kernel(input, target, code_weights):
    x = input.reshape(B, C)
    xrow = jnp.tile(x, (1, C)).reshape(1, Q)
    wrow = jnp.tile(code_weights, C * B).reshape(1, Q)
    xrep = jnp.repeat(x, C, axis=1)
    q = jnp.arange(Q, dtype=jnp.int32)
    pat = (q[None, :] % CC == jnp.arange(CC, dtype=jnp.int32)[:, None]
           ).astype(jnp.bfloat16)

    out = pl.pallas_call(
        _body,
        grid=(B // BI,),
        in_specs=[
            pl.BlockSpec((1, Q), lambda i: (0, 0)),
            pl.BlockSpec((1, Q), lambda i: (0, 0)),
            pl.BlockSpec((BI, CC), lambda i: (i, 0)),
            pl.BlockSpec((CC, Q), lambda i: (0, 0)),
        ],
        out_specs=pl.BlockSpec((BI, Q), lambda i: (i, 0)),
        out_shape=jax.ShapeDtypeStruct((B, Q), jnp.float32),
    )(xrow, wrow, xrep, pat)
    return out.reshape(B, B, C, C)
